# baseline (speedup 1.0000x reference)
import functools

import jax
import jax.numpy as jnp
from jax import lax
from jax.experimental import pallas as pl
from jax.experimental.pallas import tpu as pltpu
from jax.experimental.pallas import tpu_sc as plsc

_MARGIN = 0.5
_NON_MATCH_LOSS_WEIGHT = 1.0
_L = 16


def _sc_geometry():
    try:
        info = plsc.get_sparse_core_info()
        return info.num_cores, info.num_subcores
    except Exception:
        return 2, 16


@functools.partial(jax.jit, static_argnums=(6, 7, 8, 9))
def _partials(ta, tb, mA, mB, nA, nB, B, N, D, M):
    NC, NS = _sc_geometry()
    NW = NC * NS
    PW = M // NW
    CH = min(PW, 128)
    NCH = PW // CH
    N2 = N // 2
    G = CH // _L
    CC = D // _L
    mesh = plsc.VectorSubcoreMesh(core_axis_name="c", subcore_axis_name="s",
                                  num_cores=NC, num_subcores=NS)

    n_chunks = B * 2 * NCH

    def body(ta_hbm, tb_hbm, mA_hbm, mB_hbm, nA_hbm, nB_hbm, out_hbm,
             idxa, idxb, cola, colb, rowsa, rowsb, res_v,
             semi, semr):
        wid = lax.axis_index("s") * NC + lax.axis_index("c")
        base = wid * PW
        iota = lax.broadcasted_iota(jnp.int32, (_L,), 0)

        def chunk_desc(k):
            b, r = divmod(k, 2 * NCH)
            si, ch = divmod(r, NCH)
            return b, si, ch

        def fire_idx(k, p):
            b, si, ch = chunk_desc(k)
            iA = mA_hbm if si == 0 else nA_hbm
            iB = mB_hbm if si == 0 else nB_hbm
            start = b * M + base + ch * CH
            ca = pltpu.async_copy(iA.at[pl.ds(start, CH)], idxa[p], semi[p])
            cb = pltpu.async_copy(iB.at[pl.ds(start, CH)], idxb[p], semi[p])
            return ca, cb

        def prep(k, p):
            b, _, _ = chunk_desc(k)
            roff = jnp.int32(b * N2)
            for i in range(G):
                sl = pl.ds(i * _L, _L)
                ra = idxa[p][sl]
                cola[p][sl] = (ra & 1) << 6
                idxa[p][sl] = (ra >> 1) + roff
                rb = idxb[p][sl]
                colb[p][sl] = (rb & 1) << 6
                idxb[p][sl] = (rb >> 1) + roff

        def fire_rows(p):
            ca = pltpu.async_copy(ta_hbm.at[idxa[p]], rowsa[p], semr[p])
            cb = pltpu.async_copy(tb_hbm.at[idxb[p]], rowsb[p], semr[p])
            return ca, cb

        idx_cps = [None, None]
        row_cps = [None, None]
        idx_cps[0] = fire_idx(0, 0)
        idx_cps[1] = fire_idx(1, 1)
        idx_cps[0][0].wait()
        idx_cps[0][1].wait()
        prep(0, 0)
        row_cps[0] = fire_rows(0)

        acc_m = None
        acc_p = None
        acc_c = None

        for k in range(n_chunks):
            p = k & 1
            q = (k + 1) & 1
            b, si, ch = chunk_desc(k)
            row_cps[p][0].wait()
            row_cps[p][1].wait()
            if k + 2 < n_chunks:
                idx_cps[p] = fire_idx(k + 2, p)
            if k + 1 < n_chunks:
                idx_cps[q][0].wait()
                idx_cps[q][1].wait()
                prep(k + 1, q)
                row_cps[q] = fire_rows(q)
            if ch == 0:
                if si == 0:
                    acc_m = [jnp.zeros((_L,), jnp.float32) for _ in range(4)]
                else:
                    acc_p = [jnp.zeros((_L,), jnp.float32) for _ in range(4)]
                    acc_c = [jnp.zeros((_L,), jnp.float32) for _ in range(4)]
            UNR = 8
            if si == 0:
                def gbody_m(g, accs):
                    rows16 = g * _L + iota
                    cba = cola[p][pl.ds(g * _L, _L)]
                    cbb = colb[p][pl.ds(g * _L, _L)]
                    def cbody(it, accs):
                        accs = list(accs)
                        c0 = it * UNR
                        for j in range(UNR):
                            av = plsc.load_gather(rowsa[p], [rows16, cba + (c0 + j)])
                            bv = plsc.load_gather(rowsb[p], [rows16, cbb + (c0 + j)])
                            d = av - bv
                            accs[j & 3] = accs[j & 3] + d * d
                        return tuple(accs)
                    return lax.fori_loop(0, D // UNR, cbody, accs)
                acc_m = list(lax.fori_loop(0, G, gbody_m, tuple(acc_m)))
            else:
                def gbody_n(g, accs):
                    rows16 = g * _L + iota
                    cba = cola[p][pl.ds(g * _L, _L)]
                    cbb = colb[p][pl.ds(g * _L, _L)]
                    def cbody(it, accs):
                        a0 = list(accs[0])
                        a1 = list(accs[1])
                        c0 = it * UNR
                        for j in range(UNR):
                            av = plsc.load_gather(rowsa[p], [rows16, cba + (c0 + j)])
                            bv = plsc.load_gather(rowsb[p], [rows16, cbb + (c0 + j)])
                            d = av - bv
                            t = _MARGIN - d * d
                            pos = t > 0.0
                            a0[j & 3] = a0[j & 3] + jnp.where(pos, t, 0.0)
                            a1[j & 3] = a1[j & 3] + jnp.where(pos, 1.0, 0.0)
                        return tuple(a0), tuple(a1)
                    return lax.fori_loop(0, D // UNR, cbody, accs)
                acc_p, acc_c = lax.fori_loop(0, G, gbody_n, (tuple(acc_p), tuple(acc_c)))
                acc_p = list(acc_p)
                acc_c = list(acc_c)
            if ch == NCH - 1:
                if si == 0:
                    res_v[pl.ds(b * 64, _L)] = (acc_m[0] + acc_m[1]) + (acc_m[2] + acc_m[3])
                else:
                    res_v[pl.ds(b * 64 + _L, _L)] = (acc_p[0] + acc_p[1]) + (acc_p[2] + acc_p[3])
                    res_v[pl.ds(b * 64 + 2 * _L, _L)] = (acc_c[0] + acc_c[1]) + (acc_c[2] + acc_c[3])

        pltpu.sync_copy(res_v, out_hbm.at[pl.ds(wid * 4 * B * _L, 4 * B * _L)])

    call = pl.kernel(
        body,
        out_type=jax.ShapeDtypeStruct((NW * B * 4 * _L,), jnp.float32),
        mesh=mesh,
        scratch_types=[
            [pltpu.VMEM((CH,), jnp.int32) for _ in range(2)],
            [pltpu.VMEM((CH,), jnp.int32) for _ in range(2)],
            [pltpu.VMEM((CH,), jnp.int32) for _ in range(2)],
            [pltpu.VMEM((CH,), jnp.int32) for _ in range(2)],
            [pltpu.VMEM((CH, 2 * D), jnp.float32) for _ in range(2)],
            [pltpu.VMEM((CH, 2 * D), jnp.float32) for _ in range(2)],
            pltpu.VMEM((B * 4 * _L,), jnp.float32),
            [pltpu.SemaphoreType.DMA for _ in range(2)],
            [pltpu.SemaphoreType.DMA for _ in range(2)],
        ],
        compiler_params=pltpu.CompilerParams(needs_layout_passes=False),
    )
    return call(ta, tb, mA, mB, nA, nB)


def kernel(outA, outB, matchA, matchB, nonMatchA, nonMatchB):
    B, N, D = outA.shape
    M = matchA.shape[1]
    ta = outA.reshape(B * N // 2, 2 * D)
    tb = outB.reshape(B * N // 2, 2 * D)
    mA = matchA.astype(jnp.int32).reshape(-1)
    mB = matchB.astype(jnp.int32).reshape(-1)
    nA = nonMatchA.astype(jnp.int32).reshape(-1)
    nB = nonMatchB.astype(jnp.int32).reshape(-1)
    parts = _partials(ta, tb, mA, mB, nA, nB, B, N, D, M)
    NC, NS = _sc_geometry()
    sums = jnp.sum(parts.reshape(NC * NS, B, 4, _L), axis=(0, 3))
    match_loss = jnp.sum(sums[:, 0]) / M
    non_match_loss = _NON_MATCH_LOSS_WEIGHT * jnp.sum(sums[:, 1] / sums[:, 2])
    return (match_loss + non_match_loss, match_loss, non_match_loss)


# speedup vs baseline: 1.4576x; 1.4576x over previous
import functools

import jax
import jax.numpy as jnp
from jax import lax
from jax.experimental import pallas as pl
from jax.experimental.pallas import tpu as pltpu
from jax.experimental.pallas import tpu_sc as plsc

_MARGIN = 0.5
_NON_MATCH_LOSS_WEIGHT = 1.0
_L = 16


def _sc_geometry():
    try:
        info = plsc.get_sparse_core_info()
        return info.num_cores, info.num_subcores
    except Exception:
        return 2, 16


@functools.partial(jax.jit, static_argnums=(6, 7, 8, 9))
def _partials(ta, tb, mA, mB, nA, nB, B, N, D, M):
    NC, NS = _sc_geometry()
    NW = NC * NS
    PW = M // NW
    CH = min(PW, 128)
    NCH = PW // CH
    CC = D // _L
    G = CH // _L
    mesh = plsc.VectorSubcoreMesh(core_axis_name="c", subcore_axis_name="s",
                                  num_cores=NC, num_subcores=NS)

    n_chunks = B * 2 * NCH

    def body(ta_hbm, tb_hbm, mA_hbm, mB_hbm, nA_hbm, nB_hbm, out_hbm,
             idxa, idxb, rowsa, rowsb, res_v, semi, semr):
        wid = lax.axis_index("s") * NC + lax.axis_index("c")
        base = wid * PW

        def chunk_desc(k):
            b, r = divmod(k, 2 * NCH)
            si, ch = divmod(r, NCH)
            return b, si, ch

        def fire_idx(k, p):
            b, si, ch = chunk_desc(k)
            iA = mA_hbm if si == 0 else nA_hbm
            iB = mB_hbm if si == 0 else nB_hbm
            start = b * M + base + ch * CH
            ca = pltpu.async_copy(iA.at[pl.ds(start, CH)], idxa[p], semi[p])
            cb = pltpu.async_copy(iB.at[pl.ds(start, CH)], idxb[p], semi[p])
            return ca, cb

        def prep(k, p):
            b, _, _ = chunk_desc(k)
            roff = jnp.int32(b * N)
            for i in range(G):
                sl = pl.ds(i * _L, _L)
                idxa[p][sl] = idxa[p][sl] + roff
                idxb[p][sl] = idxb[p][sl] + roff

        def fire_rows(p):
            ca = pltpu.async_copy(ta_hbm.at[idxa[p]], rowsa[p], semr[p])
            cb = pltpu.async_copy(tb_hbm.at[idxb[p]], rowsb[p], semr[p])
            return ca, cb

        idx_cps = [None, None]
        row_cps = [None, None]
        idx_cps[0] = fire_idx(0, 0)
        idx_cps[1] = fire_idx(1, 1)
        idx_cps[0][0].wait()
        idx_cps[0][1].wait()
        prep(0, 0)
        row_cps[0] = fire_rows(0)

        acc_m = acc_p = acc_c = None

        for k in range(n_chunks):
            p = k & 1
            q = (k + 1) & 1
            b, si, ch = chunk_desc(k)
            row_cps[p][0].wait()
            row_cps[p][1].wait()
            if k + 2 < n_chunks:
                idx_cps[p] = fire_idx(k + 2, p)
            if k + 1 < n_chunks:
                idx_cps[q][0].wait()
                idx_cps[q][1].wait()
                prep(k + 1, q)
                row_cps[q] = fire_rows(q)
            if ch == 0:
                if si == 0:
                    acc_m = [jnp.zeros((_L,), jnp.float32) for _ in range(CC)]
                else:
                    acc_p = [jnp.zeros((_L,), jnp.float32) for _ in range(CC)]
                    acc_c = [jnp.zeros((_L,), jnp.float32) for _ in range(CC)]
            if si == 0:
                def rbody_m(r, accs):
                    accs = list(accs)
                    for cc in range(CC):
                        sl = pl.ds(cc * _L, _L)
                        d = rowsa[p][r, sl] - rowsb[p][r, sl]
                        accs[cc] = accs[cc] + d * d
                    return tuple(accs)
                acc_m = list(lax.fori_loop(0, CH, rbody_m, tuple(acc_m)))
            else:
                def rbody_n(r, accs):
                    a0 = list(accs[0])
                    a1 = list(accs[1])
                    for cc in range(CC):
                        sl = pl.ds(cc * _L, _L)
                        d = rowsa[p][r, sl] - rowsb[p][r, sl]
                        t = _MARGIN - d * d
                        pos = t > 0.0
                        a0[cc] = a0[cc] + jnp.where(pos, t, 0.0)
                        a1[cc] = a1[cc] + jnp.where(pos, 1.0, 0.0)
                    return tuple(a0), tuple(a1)
                acc_p, acc_c = lax.fori_loop(0, CH, rbody_n,
                                             (tuple(acc_p), tuple(acc_c)))
                acc_p = list(acc_p)
                acc_c = list(acc_c)
            if ch == NCH - 1:
                if si == 0:
                    res_v[pl.ds(b * 64, _L)] = (acc_m[0] + acc_m[1]) + (acc_m[2] + acc_m[3])
                else:
                    res_v[pl.ds(b * 64 + _L, _L)] = (acc_p[0] + acc_p[1]) + (acc_p[2] + acc_p[3])
                    res_v[pl.ds(b * 64 + 2 * _L, _L)] = (acc_c[0] + acc_c[1]) + (acc_c[2] + acc_c[3])

        pltpu.sync_copy(res_v, out_hbm.at[pl.ds(wid * 4 * B * _L, 4 * B * _L)])

    call = pl.kernel(
        body,
        out_type=jax.ShapeDtypeStruct((NW * B * 4 * _L,), jnp.float32),
        mesh=mesh,
        scratch_types=[
            [pltpu.VMEM((CH,), jnp.int32) for _ in range(2)],
            [pltpu.VMEM((CH,), jnp.int32) for _ in range(2)],
            [pltpu.VMEM((CH, D), jnp.float32) for _ in range(2)],
            [pltpu.VMEM((CH, D), jnp.float32) for _ in range(2)],
            pltpu.VMEM((B * 4 * _L,), jnp.float32),
            [pltpu.SemaphoreType.DMA for _ in range(2)],
            [pltpu.SemaphoreType.DMA for _ in range(2)],
        ],
        compiler_params=pltpu.CompilerParams(use_tc_tiling_on_sc=False),
    )
    return call(ta, tb, mA, mB, nA, nB)


def kernel(outA, outB, matchA, matchB, nonMatchA, nonMatchB):
    B, N, D = outA.shape
    M = matchA.shape[1]
    ta = outA.reshape(B * N, D)
    tb = outB.reshape(B * N, D)
    mA = matchA.astype(jnp.int32).reshape(-1)
    mB = matchB.astype(jnp.int32).reshape(-1)
    nA = nonMatchA.astype(jnp.int32).reshape(-1)
    nB = nonMatchB.astype(jnp.int32).reshape(-1)
    parts = _partials(ta, tb, mA, mB, nA, nB, B, N, D, M)
    NC, NS = _sc_geometry()
    sums = jnp.sum(parts.reshape(NC * NS, B, 4, _L), axis=(0, 3))
    match_loss = jnp.sum(sums[:, 0]) / M
    non_match_loss = _NON_MATCH_LOSS_WEIGHT * jnp.sum(sums[:, 1] / sums[:, 2])
    return (match_loss + non_match_loss, match_loss, non_match_loss)


# speedup vs baseline: 1.4591x; 1.0011x over previous
import functools

import jax
import jax.numpy as jnp
from jax import lax
from jax.experimental import pallas as pl
from jax.experimental.pallas import tpu as pltpu
from jax.experimental.pallas import tpu_sc as plsc

_MARGIN = 0.5
_NON_MATCH_LOSS_WEIGHT = 1.0
_L = 16


def _sc_geometry():
    try:
        info = plsc.get_sparse_core_info()
        return info.num_cores, info.num_subcores
    except Exception:
        return 2, 16


@functools.partial(jax.jit, static_argnums=(6, 7, 8, 9))
def _partials(ta, tb, mA, mB, nA, nB, B, N, D, M):
    NC, NS = _sc_geometry()
    NW = NC * NS
    PW = M // NW
    CH = min(PW, 128)
    NCH = PW // CH
    CC = D // _L
    G = CH // _L
    mesh = plsc.VectorSubcoreMesh(core_axis_name="c", subcore_axis_name="s",
                                  num_cores=NC, num_subcores=NS)

    n_chunks = B * 2 * NCH

    def body(ta_hbm, tb_hbm, mA_hbm, mB_hbm, nA_hbm, nB_hbm, out_hbm,
             idxa, idxb, rowsa, rowsb, res_v, semi, semr):
        wid = lax.axis_index("s") * NC + lax.axis_index("c")
        base = wid * PW

        def chunk_desc(k):
            b, r = divmod(k, 2 * NCH)
            si, ch = divmod(r, NCH)
            return b, si, ch

        def fire_idx(k, p):
            b, si, ch = chunk_desc(k)
            iA = mA_hbm if si == 0 else nA_hbm
            iB = mB_hbm if si == 0 else nB_hbm
            start = base + ch * CH
            ca = pltpu.async_copy(iA.at[b, pl.ds(start, CH)], idxa[p], semi[p])
            cb = pltpu.async_copy(iB.at[b, pl.ds(start, CH)], idxb[p], semi[p])
            return ca, cb

        def fire_rows(k, p):
            b, _, _ = chunk_desc(k)
            ca = pltpu.async_copy(ta_hbm.at[b].at[idxa[p]], rowsa[p], semr[p])
            cb = pltpu.async_copy(tb_hbm.at[b].at[idxb[p]], rowsb[p], semr[p])
            return ca, cb

        idx_cps = [None, None]
        row_cps = [None, None]
        idx_cps[0] = fire_idx(0, 0)
        idx_cps[1] = fire_idx(1, 1)
        idx_cps[0][0].wait()
        idx_cps[0][1].wait()
        row_cps[0] = fire_rows(0, 0)

        acc_m = acc_p = acc_c = None

        for k in range(n_chunks):
            p = k & 1
            q = (k + 1) & 1
            b, si, ch = chunk_desc(k)
            row_cps[p][0].wait()
            row_cps[p][1].wait()
            if k + 2 < n_chunks:
                idx_cps[p] = fire_idx(k + 2, p)
            if k + 1 < n_chunks:
                idx_cps[q][0].wait()
                idx_cps[q][1].wait()
                row_cps[q] = fire_rows(k + 1, q)
            if ch == 0:
                if si == 0:
                    acc_m = [jnp.zeros((_L,), jnp.float32) for _ in range(CC)]
                else:
                    acc_p = [jnp.zeros((_L,), jnp.float32) for _ in range(CC)]
                    acc_c = [jnp.zeros((_L,), jnp.float32) for _ in range(CC)]
            if si == 0:
                def rbody_m(r, accs):
                    accs = list(accs)
                    for cc in range(CC):
                        sl = pl.ds(cc * _L, _L)
                        d = rowsa[p][r, sl] - rowsb[p][r, sl]
                        accs[cc] = accs[cc] + d * d
                    return tuple(accs)
                acc_m = list(lax.fori_loop(0, CH, rbody_m, tuple(acc_m)))
            else:
                def rbody_n(r, accs):
                    a0 = list(accs[0])
                    a1 = list(accs[1])
                    for cc in range(CC):
                        sl = pl.ds(cc * _L, _L)
                        d = rowsa[p][r, sl] - rowsb[p][r, sl]
                        t = _MARGIN - d * d
                        pos = t > 0.0
                        a0[cc] = a0[cc] + jnp.where(pos, t, 0.0)
                        a1[cc] = a1[cc] + jnp.where(pos, 1.0, 0.0)
                    return tuple(a0), tuple(a1)
                acc_p, acc_c = lax.fori_loop(0, CH, rbody_n,
                                             (tuple(acc_p), tuple(acc_c)))
                acc_p = list(acc_p)
                acc_c = list(acc_c)
            if ch == NCH - 1:
                if si == 0:
                    res_v[pl.ds(b * 64, _L)] = (acc_m[0] + acc_m[1]) + (acc_m[2] + acc_m[3])
                else:
                    res_v[pl.ds(b * 64 + _L, _L)] = (acc_p[0] + acc_p[1]) + (acc_p[2] + acc_p[3])
                    res_v[pl.ds(b * 64 + 2 * _L, _L)] = (acc_c[0] + acc_c[1]) + (acc_c[2] + acc_c[3])

        pltpu.sync_copy(res_v, out_hbm.at[pl.ds(wid * 4 * B * _L, 4 * B * _L)])

    call = pl.kernel(
        body,
        out_type=jax.ShapeDtypeStruct((NW * B * 4 * _L,), jnp.float32),
        mesh=mesh,
        scratch_types=[
            [pltpu.VMEM((CH,), jnp.int32) for _ in range(2)],
            [pltpu.VMEM((CH,), jnp.int32) for _ in range(2)],
            [pltpu.VMEM((CH, D), jnp.float32) for _ in range(2)],
            [pltpu.VMEM((CH, D), jnp.float32) for _ in range(2)],
            pltpu.VMEM((B * 4 * _L,), jnp.float32),
            [pltpu.SemaphoreType.DMA for _ in range(2)],
            [pltpu.SemaphoreType.DMA for _ in range(2)],
        ],
        compiler_params=pltpu.CompilerParams(use_tc_tiling_on_sc=False),
    )
    return call(ta, tb, mA, mB, nA, nB)


def kernel(outA, outB, matchA, matchB, nonMatchA, nonMatchB):
    B, N, D = outA.shape
    M = matchA.shape[1]
    mA = matchA.astype(jnp.int32)
    mB = matchB.astype(jnp.int32)
    nA = nonMatchA.astype(jnp.int32)
    nB = nonMatchB.astype(jnp.int32)
    parts = _partials(outA, outB, mA, mB, nA, nB, B, N, D, M)
    NC, NS = _sc_geometry()
    sums = jnp.sum(parts.reshape(NC * NS, B, 4, _L), axis=(0, 3))
    match_loss = jnp.sum(sums[:, 0]) / M
    non_match_loss = _NON_MATCH_LOSS_WEIGHT * jnp.sum(sums[:, 1] / sums[:, 2])
    return (match_loss + non_match_loss, match_loss, non_match_loss)
